# baseline (speedup 1.0000x reference)
import functools

import jax
import jax.numpy as jnp
from jax.experimental import pallas as pl


N = 4096
BM = 256


def _xw0_body(x_ref, w0_ref, out_ref):
    out_ref[...] = jnp.dot(
        x_ref[...], w0_ref[...], preferred_element_type=jnp.float32
    )


def _layer1_body(a_ref, xw0_ref, b0_ref, w1_ref, out_ref):
    h = jnp.dot(a_ref[...], xw0_ref[...], preferred_element_type=jnp.float32)
    h = jnp.maximum(h + b0_ref[...], 0.0)
    out_ref[...] = jnp.dot(h, w1_ref[...], preferred_element_type=jnp.float32)


def _layer2_body(a_ref, p_ref, b1_ref, out_ref):
    g = jnp.dot(a_ref[...], p_ref[...], preferred_element_type=jnp.float32)
    g = g + b1_ref[...]
    m = jnp.max(g, axis=-1, keepdims=True)
    s = g - m
    lse = jnp.log(jnp.sum(jnp.exp(s), axis=-1, keepdims=True))
    out_ref[...] = s - lse


@jax.jit
def kernel(x, adjs, W0, b0, W1, b1):
    fin = x.shape[1]
    h_dim = W0.shape[1]
    fout = W1.shape[1]
    a0 = adjs[0]
    a1 = adjs[1]
    b0r = b0.reshape(1, h_dim)
    b1r = b1.reshape(1, fout)

    xw0 = pl.pallas_call(
        _xw0_body,
        out_shape=jax.ShapeDtypeStruct((N, h_dim), jnp.float32),
        in_specs=[
            pl.BlockSpec((N, fin), lambda: (0, 0)),
            pl.BlockSpec((fin, h_dim), lambda: (0, 0)),
        ],
        out_specs=pl.BlockSpec((N, h_dim), lambda: (0, 0)),
    )(x, W0)

    p = pl.pallas_call(
        _layer1_body,
        grid=(N // BM,),
        out_shape=jax.ShapeDtypeStruct((N, fout), jnp.float32),
        in_specs=[
            pl.BlockSpec((BM, N), lambda i: (i, 0)),
            pl.BlockSpec((N, h_dim), lambda i: (0, 0)),
            pl.BlockSpec((1, h_dim), lambda i: (0, 0)),
            pl.BlockSpec((h_dim, fout), lambda i: (0, 0)),
        ],
        out_specs=pl.BlockSpec((BM, fout), lambda i: (i, 0)),
    )(a0, xw0, b0r, W1)

    out = pl.pallas_call(
        _layer2_body,
        grid=(N // BM,),
        out_shape=jax.ShapeDtypeStruct((N, fout), jnp.float32),
        in_specs=[
            pl.BlockSpec((BM, N), lambda i: (i, 0)),
            pl.BlockSpec((N, fout), lambda i: (0, 0)),
            pl.BlockSpec((1, fout), lambda i: (0, 0)),
        ],
        out_specs=pl.BlockSpec((BM, fout), lambda i: (i, 0)),
    )(a1, p, b1r)

    return out


# speedup vs baseline: 1.0141x; 1.0141x over previous
import functools

import jax
import jax.numpy as jnp
from jax.experimental import pallas as pl


N = 4096
BM = 256


def _xw0_body(x_ref, w0_ref, out_ref):
    out_ref[...] = jnp.dot(
        x_ref[...], w0_ref[...], preferred_element_type=jnp.float32
    ).astype(jnp.bfloat16)


def _layer1_body(a_ref, xw0_ref, b0_ref, w1_ref, out_ref):
    a = a_ref[...].astype(jnp.bfloat16)
    h = jnp.dot(a, xw0_ref[...], preferred_element_type=jnp.float32)
    h = jnp.maximum(h + b0_ref[...], 0.0)
    out_ref[...] = jnp.dot(
        h, w1_ref[...], preferred_element_type=jnp.float32
    ).astype(jnp.bfloat16)


def _layer2_body(a_ref, p_ref, b1_ref, out_ref):
    a = a_ref[...].astype(jnp.bfloat16)
    g = jnp.dot(a, p_ref[...], preferred_element_type=jnp.float32)
    g = g + b1_ref[...]
    m = jnp.max(g, axis=-1, keepdims=True)
    s = g - m
    lse = jnp.log(jnp.sum(jnp.exp(s), axis=-1, keepdims=True))
    out_ref[...] = s - lse


@jax.jit
def kernel(x, adjs, W0, b0, W1, b1):
    fin = x.shape[1]
    h_dim = W0.shape[1]
    fout = W1.shape[1]
    a0 = adjs[0]
    a1 = adjs[1]
    b0r = b0.reshape(1, h_dim)
    b1r = b1.reshape(1, fout)

    xw0 = pl.pallas_call(
        _xw0_body,
        out_shape=jax.ShapeDtypeStruct((N, h_dim), jnp.bfloat16),
        in_specs=[
            pl.BlockSpec((N, fin), lambda: (0, 0)),
            pl.BlockSpec((fin, h_dim), lambda: (0, 0)),
        ],
        out_specs=pl.BlockSpec((N, h_dim), lambda: (0, 0)),
    )(x, W0)

    p = pl.pallas_call(
        _layer1_body,
        grid=(N // BM,),
        out_shape=jax.ShapeDtypeStruct((N, fout), jnp.bfloat16),
        in_specs=[
            pl.BlockSpec((BM, N), lambda i: (i, 0)),
            pl.BlockSpec((N, h_dim), lambda i: (0, 0)),
            pl.BlockSpec((1, h_dim), lambda i: (0, 0)),
            pl.BlockSpec((h_dim, fout), lambda i: (0, 0)),
        ],
        out_specs=pl.BlockSpec((BM, fout), lambda i: (i, 0)),
    )(a0, xw0, b0r, W1)

    out = pl.pallas_call(
        _layer2_body,
        grid=(N // BM,),
        out_shape=jax.ShapeDtypeStruct((N, fout), jnp.float32),
        in_specs=[
            pl.BlockSpec((BM, N), lambda i: (i, 0)),
            pl.BlockSpec((N, fout), lambda i: (0, 0)),
            pl.BlockSpec((1, fout), lambda i: (0, 0)),
        ],
        out_specs=pl.BlockSpec((BM, fout), lambda i: (i, 0)),
    )(a1, p, b1r)

    return out


# speedup vs baseline: 2.4916x; 2.4568x over previous
import jax
import jax.numpy as jnp
from jax.experimental import pallas as pl
from jax.experimental.pallas import tpu as pltpu


N = 4096
BM = 256


def _layer1_body(a_ref, x_ref, w0_ref, b0_ref, w1_ref, out_ref, xw0_ref):
    @pl.when(pl.program_id(0) == 0)
    def _():
        xw0_ref[...] = jnp.dot(
            x_ref[...], w0_ref[...], preferred_element_type=jnp.float32
        ).astype(jnp.bfloat16)

    a = a_ref[0].astype(jnp.bfloat16)
    h = jnp.dot(a, xw0_ref[...], preferred_element_type=jnp.float32)
    h = jnp.maximum(h + b0_ref[...], 0.0)
    out_ref[...] = jnp.dot(
        h, w1_ref[...], preferred_element_type=jnp.float32
    ).astype(jnp.bfloat16)


def _layer2_body(a_ref, p_ref, b1_ref, out_ref):
    a = a_ref[0].astype(jnp.bfloat16)
    g = jnp.dot(a, p_ref[...], preferred_element_type=jnp.float32)
    g = g + b1_ref[...]
    m = jnp.max(g, axis=-1, keepdims=True)
    s = g - m
    lse = jnp.log(jnp.sum(jnp.exp(s), axis=-1, keepdims=True))
    out_ref[...] = s - lse


@jax.jit
def kernel(x, adjs, W0, b0, W1, b1):
    fin = x.shape[1]
    h_dim = W0.shape[1]
    fout = W1.shape[1]
    b0r = b0.reshape(1, h_dim)
    b1r = b1.reshape(1, fout)

    p = pl.pallas_call(
        _layer1_body,
        grid=(N // BM,),
        out_shape=jax.ShapeDtypeStruct((N, fout), jnp.bfloat16),
        in_specs=[
            pl.BlockSpec((1, BM, N), lambda i: (0, i, 0)),
            pl.BlockSpec((N, fin), lambda i: (0, 0)),
            pl.BlockSpec((fin, h_dim), lambda i: (0, 0)),
            pl.BlockSpec((1, h_dim), lambda i: (0, 0)),
            pl.BlockSpec((h_dim, fout), lambda i: (0, 0)),
        ],
        out_specs=pl.BlockSpec((BM, fout), lambda i: (i, 0)),
        scratch_shapes=[pltpu.VMEM((N, h_dim), jnp.bfloat16)],
    )(adjs, x, W0, b0r, W1)

    out = pl.pallas_call(
        _layer2_body,
        grid=(N // BM,),
        out_shape=jax.ShapeDtypeStruct((N, fout), jnp.float32),
        in_specs=[
            pl.BlockSpec((1, BM, N), lambda i: (1, i, 0)),
            pl.BlockSpec((N, fout), lambda i: (0, 0)),
            pl.BlockSpec((1, fout), lambda i: (0, 0)),
        ],
        out_specs=pl.BlockSpec((BM, fout), lambda i: (i, 0)),
    )(adjs, p, b1r)

    return out
